# baseline (speedup 1.0000x reference)
import functools

import jax
import jax.numpy as jnp
import numpy as np
from jax import lax
from jax.experimental import pallas as pl
from jax.experimental.pallas import tpu as pltpu
from jax.experimental.pallas import tpu_sc as plsc

N = 16384
EMB = 128
H2 = 512
OUT = 256
NW = 32
RPW = N // NW
IDX_W = 128
CHUNKS = RPW // IDX_W
SCL = 16
BLK = 8192

_sc_mesh = plsc.VectorSubcoreMesh(core_axis_name="c", subcore_axis_name="s")


VOCAB = 1000
EMBW = EMB // 2


@functools.partial(
    pl.kernel,
    mesh=_sc_mesh,
    out_type=(
        jax.ShapeDtypeStruct((N, EMBW), jnp.int32),
        jax.ShapeDtypeStruct((N, EMBW), jnp.int32),
    ),
    scratch_types=[
        pltpu.VMEM((CHUNKS, IDX_W), jnp.int32),
        pltpu.VMEM((RPW, EMBW), jnp.int32),
        pltpu.VMEM_SHARED((VOCAB, EMBW), jnp.int32),
        pltpu.VMEM_SHARED((VOCAB, EMBW), jnp.int32),
        pltpu.SemaphoreType.DMA,
    ],
)
def _sc_gather(gtab, ktab, gidx, kidx, gout, kout, idx_v, rows_v, spm_g,
               spm_k, sem):
    wid = lax.axis_index("s") * 2 + lax.axis_index("c")
    row0 = wid * RPW
    blk0 = wid * CHUNKS

    @pl.when(lax.axis_index("s") == 0)
    def _stage():
        pltpu.sync_copy(gtab, spm_g)
        pltpu.sync_copy(ktab, spm_k)

    plsc.subcore_barrier()

    def one_table(tab, out_hbm, idx_hbm):
        pltpu.sync_copy(idx_hbm.at[pl.ds(blk0, CHUNKS)], idx_v)
        copies = []
        for j in range(CHUNKS):
            copies.append(
                pltpu.async_copy(
                    tab.at[idx_v.at[j]], rows_v.at[pl.ds(j * IDX_W, IDX_W)], sem
                )
            )
        for c in copies:
            c.wait()
        pltpu.sync_copy(rows_v, out_hbm.at[pl.ds(row0, RPW)])

    one_table(spm_g, gout, gidx)
    one_table(spm_k, kout, kidx)


def _pack_tables_body(g_ref, k_ref, gw_ref, kw_ref):
    for src, dst in ((g_ref, gw_ref), (k_ref, kw_ref)):
        bits = lax.bitcast_convert_type(
            src[...].astype(jnp.bfloat16), jnp.int16
        )
        lo = bits[:, :EMBW].astype(jnp.int32) & jnp.int32(0xFFFF)
        hi = bits[:, EMBW:].astype(jnp.int32) << 16
        dst[...] = lo | hi


def _pack_tables(gtab, ktab):
    return pl.pallas_call(
        _pack_tables_body,
        out_shape=(
            jax.ShapeDtypeStruct((VOCAB, EMBW), jnp.int32),
            jax.ShapeDtypeStruct((VOCAB, EMBW), jnp.int32),
        ),
    )(gtab, ktab)


def _unpack_halves(x):
    bf = jnp.bfloat16
    lo = lax.bitcast_convert_type(x << 16, jnp.float32).astype(bf)
    hi = lax.bitcast_convert_type(
        x & jnp.int32(-65536), jnp.float32
    ).astype(bf)
    return lo, hi


def _mlp_body(g_ref, k_ref, w1_ref, w2_ref, b1_ref, b2_ref, o_ref):
    dnums = (((1,), (1,)), ((), ()))
    bf = jnp.bfloat16
    g_lo, g_hi = _unpack_halves(g_ref[...])
    k_lo, k_hi = _unpack_halves(k_ref[...])
    emb = jnp.concatenate([g_lo, g_hi, k_lo, k_hi], axis=1)
    h = lax.dot_general(
        emb, w1_ref[...].astype(bf), dnums, preferred_element_type=jnp.float32
    )
    h16 = jnp.maximum(h.astype(bf) + b1_ref[...].astype(bf), jnp.asarray(0, bf))
    o_ref[...] = (
        lax.dot_general(
            h16, w2_ref[...].astype(bf), dnums, preferred_element_type=jnp.float32
        )
        + b2_ref[...]
    )


def _mlp(gbuf, kbuf, w1, w2, b1, b2):
    return pl.pallas_call(
        _mlp_body,
        grid=(N // BLK,),
        in_specs=[
            pl.BlockSpec((BLK, EMBW), lambda i: (i, 0)),
            pl.BlockSpec((BLK, EMBW), lambda i: (i, 0)),
            pl.BlockSpec((H2, 2 * EMB), lambda i: (0, 0)),
            pl.BlockSpec((OUT, H2), lambda i: (0, 0)),
            pl.BlockSpec((1, H2), lambda i: (0, 0)),
            pl.BlockSpec((1, OUT), lambda i: (0, 0)),
        ],
        out_specs=pl.BlockSpec((BLK, OUT), lambda i: (i, 0)),
        out_shape=jax.ShapeDtypeStruct((N, OUT), jnp.float32),
    )(gbuf, kbuf, w1, w2, b1, b2)


def kernel(input_tensor, genre_table, key_table, W1, b1, W2, b2):
    g_idx = input_tensor[:, 0].reshape(N // IDX_W, IDX_W)
    k_idx = input_tensor[:, 1].reshape(N // IDX_W, IDX_W)
    gw, kw = _pack_tables(genre_table, key_table)
    gbuf, kbuf = _sc_gather(gw, kw, g_idx, k_idx)
    return _mlp(gbuf, kbuf, W1, W2, b1.reshape(1, H2), b2.reshape(1, OUT))


# speedup vs baseline: 1.0535x; 1.0535x over previous
import functools

import jax
import jax.numpy as jnp
import numpy as np
from jax import lax
from jax.experimental import pallas as pl
from jax.experimental.pallas import tpu as pltpu
from jax.experimental.pallas import tpu_sc as plsc

N = 16384
EMB = 128
H2 = 512
OUT = 256
NW = 32
RPW = N // NW
IDX_W = 128
CHUNKS = RPW // IDX_W
SCL = 16
BLK = 4096

_sc_mesh = plsc.VectorSubcoreMesh(core_axis_name="c", subcore_axis_name="s")


VOCAB = 1000
EMBW = EMB // 2


@functools.partial(
    pl.kernel,
    mesh=_sc_mesh,
    out_type=(
        jax.ShapeDtypeStruct((N, EMBW), jnp.int32),
        jax.ShapeDtypeStruct((N, EMBW), jnp.int32),
    ),
    scratch_types=[
        pltpu.VMEM((2, CHUNKS, IDX_W), jnp.int32),
        pltpu.VMEM((3, IDX_W, EMBW), jnp.int32),
        pltpu.VMEM_SHARED((VOCAB, EMBW), jnp.int32),
        pltpu.VMEM_SHARED((VOCAB, EMBW), jnp.int32),
        pltpu.SemaphoreType.DMA,
        pltpu.SemaphoreType.DMA,
        pltpu.SemaphoreType.DMA,
        pltpu.SemaphoreType.DMA,
        pltpu.SemaphoreType.DMA,
        pltpu.SemaphoreType.DMA,
    ],
)
def _sc_gather(gtab, ktab, gidx, kidx, gout, kout, idx_v, ring, spm_g,
               spm_k, gs0, gs1, gs2, ss0, ss1, ss2):
    NBUF = 3
    NCH = 2 * CHUNKS
    gsems = (gs0, gs1, gs2)
    ssems = (ss0, ss1, ss2)
    wid = lax.axis_index("s") * 2 + lax.axis_index("c")
    row0 = wid * RPW
    blk0 = wid * CHUNKS

    pltpu.sync_copy(gidx.at[pl.ds(blk0, CHUNKS)], idx_v.at[0])
    pltpu.sync_copy(kidx.at[pl.ds(blk0, CHUNKS)], idx_v.at[1])

    @pl.when(lax.axis_index("s") == 0)
    def _stage():
        pltpu.sync_copy(gtab, spm_g)
        pltpu.sync_copy(ktab, spm_k)

    plsc.subcore_barrier()

    tabs = (spm_g, spm_k)
    outs = (gout, kout)
    gathers = [None] * NCH
    stores = [None] * NCH

    def fire_gather(i):
        t, j = divmod(i, CHUNKS)
        return pltpu.async_copy(
            tabs[t].at[idx_v.at[t, j]], ring.at[i % NBUF], gsems[i % NBUF]
        )

    def fire_store(i):
        t, j = divmod(i, CHUNKS)
        return pltpu.async_copy(
            ring.at[i % NBUF],
            outs[t].at[pl.ds(row0 + j * IDX_W, IDX_W)],
            ssems[i % NBUF],
        )

    for i in range(NCH):
        if i >= NBUF:
            stores[i - NBUF].wait()
        gathers[i] = fire_gather(i)
        ii = i - (NBUF - 1)
        if ii >= 0:
            gathers[ii].wait()
            stores[ii] = fire_store(ii)
    for ii in range(NCH - (NBUF - 1), NCH):
        gathers[ii].wait()
        stores[ii] = fire_store(ii)
    for ii in range(NCH - NBUF, NCH):
        stores[ii].wait()


def _pack_tables_body(g_ref, k_ref, gw_ref, kw_ref):
    for src, dst in ((g_ref, gw_ref), (k_ref, kw_ref)):
        bits = lax.bitcast_convert_type(
            src[...].astype(jnp.bfloat16), jnp.int16
        )
        lo = bits[:, :EMBW].astype(jnp.int32) & jnp.int32(0xFFFF)
        hi = bits[:, EMBW:].astype(jnp.int32) << 16
        dst[...] = lo | hi


def _pack_tables(gtab, ktab):
    return pl.pallas_call(
        _pack_tables_body,
        out_shape=(
            jax.ShapeDtypeStruct((VOCAB, EMBW), jnp.int32),
            jax.ShapeDtypeStruct((VOCAB, EMBW), jnp.int32),
        ),
    )(gtab, ktab)


def _unpack_halves(x):
    bf = jnp.bfloat16
    lo = lax.bitcast_convert_type(x << 16, jnp.float32).astype(bf)
    hi = lax.bitcast_convert_type(
        x & jnp.int32(-65536), jnp.float32
    ).astype(bf)
    return lo, hi


def _mlp_body(g_ref, k_ref, w1_ref, w2_ref, b1_ref, b2_ref, o_ref):
    dnums = (((1,), (1,)), ((), ()))
    bf = jnp.bfloat16
    g_lo, g_hi = _unpack_halves(g_ref[...])
    k_lo, k_hi = _unpack_halves(k_ref[...])
    emb = jnp.concatenate([g_lo, g_hi, k_lo, k_hi], axis=1)
    h = lax.dot_general(
        emb, w1_ref[...].astype(bf), dnums, preferred_element_type=jnp.float32
    )
    h16 = jnp.maximum(h.astype(bf) + b1_ref[...].astype(bf), jnp.asarray(0, bf))
    o_ref[...] = (
        lax.dot_general(
            h16, w2_ref[...].astype(bf), dnums, preferred_element_type=jnp.float32
        )
        + b2_ref[...]
    )


def _mlp(gbuf, kbuf, w1, w2, b1, b2):
    return pl.pallas_call(
        _mlp_body,
        grid=(N // BLK,),
        in_specs=[
            pl.BlockSpec((BLK, EMBW), lambda i: (i, 0)),
            pl.BlockSpec((BLK, EMBW), lambda i: (i, 0)),
            pl.BlockSpec((H2, 2 * EMB), lambda i: (0, 0)),
            pl.BlockSpec((OUT, H2), lambda i: (0, 0)),
            pl.BlockSpec((1, H2), lambda i: (0, 0)),
            pl.BlockSpec((1, OUT), lambda i: (0, 0)),
        ],
        out_specs=pl.BlockSpec((BLK, OUT), lambda i: (i, 0)),
        out_shape=jax.ShapeDtypeStruct((N, OUT), jnp.float32),
    )(gbuf, kbuf, w1, w2, b1, b2)


def kernel(input_tensor, genre_table, key_table, W1, b1, W2, b2):
    g_idx = input_tensor[:, 0].reshape(N // IDX_W, IDX_W)
    k_idx = input_tensor[:, 1].reshape(N // IDX_W, IDX_W)
    gw, kw = _pack_tables(genre_table, key_table)
    gbuf, kbuf = _sc_gather(gw, kw, g_idx, k_idx)
    return _mlp(gbuf, kbuf, W1, W2, b1.reshape(1, H2), b2.reshape(1, OUT))


# speedup vs baseline: 1.0704x; 1.0160x over previous
import functools

import jax
import jax.numpy as jnp
import numpy as np
from jax import lax
from jax.experimental import pallas as pl
from jax.experimental.pallas import tpu as pltpu
from jax.experimental.pallas import tpu_sc as plsc

N = 16384
EMB = 128
H2 = 512
OUT = 256
NW = 32
RPW = N // NW
IDX_W = 128
CHUNKS = RPW // IDX_W
SCL = 16
BLK = 4096

_sc_mesh = plsc.VectorSubcoreMesh(core_axis_name="c", subcore_axis_name="s")


VOCAB = 1000
EMBW = EMB // 2


@functools.partial(
    pl.kernel,
    mesh=_sc_mesh,
    out_type=(
        jax.ShapeDtypeStruct((N, EMBW), jnp.int32),
        jax.ShapeDtypeStruct((N, EMBW), jnp.int32),
    ),
    scratch_types=[
        pltpu.VMEM((2, RPW), jnp.int32),
        pltpu.VMEM((4, IDX_W, EMBW), jnp.int32),
        pltpu.VMEM_SHARED((VOCAB, EMBW), jnp.int32),
        pltpu.VMEM_SHARED((VOCAB, EMBW), jnp.int32),
        pltpu.SemaphoreType.DMA,
        pltpu.SemaphoreType.DMA,
        pltpu.SemaphoreType.DMA,
        pltpu.SemaphoreType.DMA,
        pltpu.SemaphoreType.DMA,
        pltpu.SemaphoreType.DMA,
        pltpu.SemaphoreType.DMA,
        pltpu.SemaphoreType.DMA,
    ],
)
def _sc_gather(gtab, ktab, gidx, kidx, gout, kout, idx_v, ring, spm_g,
               spm_k, gs0, gs1, gs2, gs3, ss0, ss1, ss2, ss3):
    NBUF = 4
    NCH = 2 * CHUNKS
    gsems = (gs0, gs1, gs2, gs3)
    ssems = (ss0, ss1, ss2, ss3)
    sid = lax.axis_index("s")
    wid = sid * 2 + lax.axis_index("c")
    row0 = wid * RPW

    pltpu.sync_copy(gidx.at[pl.ds(row0, RPW)], idx_v.at[0])
    pltpu.sync_copy(kidx.at[pl.ds(row0, RPW)], idx_v.at[1])

    @pl.when(sid == 0)
    def _stage_g():
        pltpu.sync_copy(gtab, spm_g)

    @pl.when(sid == 1)
    def _stage_k():
        pltpu.sync_copy(ktab, spm_k)

    plsc.subcore_barrier()

    tabs = (spm_g, spm_k)
    outs = (gout, kout)
    gathers = [None] * NCH
    stores = [None] * NCH

    def fire_gather(i):
        t, j = divmod(i, CHUNKS)
        return pltpu.async_copy(
            tabs[t].at[idx_v.at[t, pl.ds(j * IDX_W, IDX_W)]],
            ring.at[i % NBUF],
            gsems[i % NBUF],
        )

    def fire_store(i):
        t, j = divmod(i, CHUNKS)
        return pltpu.async_copy(
            ring.at[i % NBUF],
            outs[t].at[pl.ds(row0 + j * IDX_W, IDX_W)],
            ssems[i % NBUF],
        )

    for i in range(NCH):
        if i >= NBUF:
            stores[i - NBUF].wait()
        gathers[i] = fire_gather(i)
        ii = i - (NBUF - 1)
        if ii >= 0:
            gathers[ii].wait()
            stores[ii] = fire_store(ii)
    for ii in range(NCH - (NBUF - 1), NCH):
        gathers[ii].wait()
        stores[ii] = fire_store(ii)
    for ii in range(NCH - NBUF, NCH):
        stores[ii].wait()


def _pack_tables_body(g_ref, k_ref, gw_ref, kw_ref):
    for src, dst in ((g_ref, gw_ref), (k_ref, kw_ref)):
        bits = lax.bitcast_convert_type(
            src[...].astype(jnp.bfloat16), jnp.int16
        )
        lo = bits[:, :EMBW].astype(jnp.int32) & jnp.int32(0xFFFF)
        hi = bits[:, EMBW:].astype(jnp.int32) << 16
        dst[...] = lo | hi


def _pack_tables(gtab, ktab):
    return pl.pallas_call(
        _pack_tables_body,
        out_shape=(
            jax.ShapeDtypeStruct((VOCAB, EMBW), jnp.int32),
            jax.ShapeDtypeStruct((VOCAB, EMBW), jnp.int32),
        ),
    )(gtab, ktab)


def _unpack_halves(x):
    bf = jnp.bfloat16
    lo = lax.bitcast_convert_type(x << 16, jnp.float32).astype(bf)
    hi = lax.bitcast_convert_type(
        x & jnp.int32(-65536), jnp.float32
    ).astype(bf)
    return lo, hi


def _mlp_body(g_ref, k_ref, w1_ref, w2_ref, b1_ref, b2_ref, o_ref):
    dnums = (((1,), (1,)), ((), ()))
    bf = jnp.bfloat16
    g_lo, g_hi = _unpack_halves(g_ref[...])
    k_lo, k_hi = _unpack_halves(k_ref[...])
    emb = jnp.concatenate([g_lo, g_hi, k_lo, k_hi], axis=1)
    h = lax.dot_general(
        emb, w1_ref[...].astype(bf), dnums, preferred_element_type=jnp.float32
    )
    h16 = jnp.maximum(h.astype(bf) + b1_ref[...].astype(bf), jnp.asarray(0, bf))
    o_ref[...] = (
        lax.dot_general(
            h16, w2_ref[...].astype(bf), dnums, preferred_element_type=jnp.float32
        )
        + b2_ref[...]
    )


def _mlp(gbuf, kbuf, w1, w2, b1, b2):
    return pl.pallas_call(
        _mlp_body,
        grid=(N // BLK,),
        in_specs=[
            pl.BlockSpec((BLK, EMBW), lambda i: (i, 0)),
            pl.BlockSpec((BLK, EMBW), lambda i: (i, 0)),
            pl.BlockSpec((H2, 2 * EMB), lambda i: (0, 0)),
            pl.BlockSpec((OUT, H2), lambda i: (0, 0)),
            pl.BlockSpec((1, H2), lambda i: (0, 0)),
            pl.BlockSpec((1, OUT), lambda i: (0, 0)),
        ],
        out_specs=pl.BlockSpec((BLK, OUT), lambda i: (i, 0)),
        out_shape=jax.ShapeDtypeStruct((N, OUT), jnp.float32),
    )(gbuf, kbuf, w1, w2, b1, b2)


def kernel(input_tensor, genre_table, key_table, W1, b1, W2, b2):
    g_idx = input_tensor[:, 0]
    k_idx = input_tensor[:, 1]
    gw, kw = _pack_tables(genre_table, key_table)
    gbuf, kbuf = _sc_gather(gw, kw, g_idx, k_idx)
    return _mlp(gbuf, kbuf, W1, W2, b1.reshape(1, H2), b2.reshape(1, OUT))


# speedup vs baseline: 1.0714x; 1.0010x over previous
import functools

import jax
import jax.numpy as jnp
import numpy as np
from jax import lax
from jax.experimental import pallas as pl
from jax.experimental.pallas import tpu as pltpu
from jax.experimental.pallas import tpu_sc as plsc

N = 16384
EMB = 128
H2 = 512
OUT = 256
NW = 32
RPW = N // NW
IDX_W = 128
CHUNKS = RPW // IDX_W
SCL = 16
BLK = 4096

_sc_mesh = plsc.VectorSubcoreMesh(core_axis_name="c", subcore_axis_name="s")


VOCAB = 1000
EMBW = EMB // 2


@functools.partial(
    pl.kernel,
    mesh=_sc_mesh,
    out_type=(
        jax.ShapeDtypeStruct((N, EMBW), jnp.int32),
        jax.ShapeDtypeStruct((N, EMBW), jnp.int32),
    ),
    scratch_types=[
        pltpu.VMEM((2, RPW), jnp.int32),
        pltpu.VMEM((4, IDX_W, EMBW), jnp.int32),
        pltpu.VMEM_SHARED((VOCAB, EMBW), jnp.int32),
        pltpu.VMEM_SHARED((VOCAB, EMBW), jnp.int32),
        pltpu.SemaphoreType.DMA,
        pltpu.SemaphoreType.DMA,
        pltpu.SemaphoreType.DMA,
        pltpu.SemaphoreType.DMA,
        pltpu.SemaphoreType.DMA,
        pltpu.SemaphoreType.DMA,
        pltpu.SemaphoreType.DMA,
        pltpu.SemaphoreType.DMA,
    ],
)
def _sc_gather(gtab, ktab, gidx, kidx, gout, kout, idx_v, ring, spm_g,
               spm_k, gs0, gs1, gs2, gs3, ss0, ss1, ss2, ss3):
    NBUF = 4
    NCH = 2 * CHUNKS
    gsems = (gs0, gs1, gs2, gs3)
    ssems = (ss0, ss1, ss2, ss3)
    sid = lax.axis_index("s")
    wid = sid * 2 + lax.axis_index("c")
    row0 = wid * RPW

    pltpu.sync_copy(gidx.at[pl.ds(row0, RPW)], idx_v.at[0])
    pltpu.sync_copy(kidx.at[pl.ds(row0, RPW)], idx_v.at[1])

    @pl.when(sid == 0)
    def _stage_g():
        pltpu.sync_copy(gtab, spm_g)

    @pl.when(sid == 1)
    def _stage_k():
        pltpu.sync_copy(ktab, spm_k)

    plsc.subcore_barrier()

    tabs = (spm_g, spm_k)
    outs = (gout, kout)
    gathers = [None] * NCH
    stores = [None] * NCH

    def fire_gather(i):
        t, j = divmod(i, CHUNKS)
        return pltpu.async_copy(
            tabs[t].at[idx_v.at[t, pl.ds(j * IDX_W, IDX_W)]],
            ring.at[i % NBUF],
            gsems[i % NBUF],
        )

    def fire_store(i):
        t, j = divmod(i, CHUNKS)
        return pltpu.async_copy(
            ring.at[i % NBUF],
            outs[t].at[pl.ds(row0 + j * IDX_W, IDX_W)],
            ssems[i % NBUF],
        )

    for i in range(NCH):
        if i >= NBUF:
            stores[i - NBUF].wait()
        gathers[i] = fire_gather(i)
        ii = i - (NBUF - 1)
        if ii >= 0:
            gathers[ii].wait()
            stores[ii] = fire_store(ii)
    for ii in range(NCH - (NBUF - 1), NCH):
        gathers[ii].wait()
        stores[ii] = fire_store(ii)
    for ii in range(NCH - NBUF, NCH):
        stores[ii].wait()


def _pack_tables_body(g_ref, k_ref, gw_ref, kw_ref):
    for src, dst in ((g_ref, gw_ref), (k_ref, kw_ref)):
        bits = lax.bitcast_convert_type(
            src[...].astype(jnp.bfloat16), jnp.int16
        )
        lo = bits[:, :EMBW].astype(jnp.int32) & jnp.int32(0xFFFF)
        hi = bits[:, EMBW:].astype(jnp.int32) << 16
        dst[...] = lo | hi


def _pack_tables(gtab, ktab):
    return pl.pallas_call(
        _pack_tables_body,
        out_shape=(
            jax.ShapeDtypeStruct((VOCAB, EMBW), jnp.int32),
            jax.ShapeDtypeStruct((VOCAB, EMBW), jnp.int32),
        ),
    )(gtab, ktab)


def _unpack_halves(x):
    bf = jnp.bfloat16
    lo = lax.bitcast_convert_type(x << 16, jnp.float32).astype(bf)
    hi = lax.bitcast_convert_type(
        x & jnp.int32(-65536), jnp.float32
    ).astype(bf)
    return lo, hi


def _mlp_body(g_ref, k_ref, w1_ref, w2_ref, b1_ref, b2_ref, o_ref):
    dnums = (((1,), (1,)), ((), ()))
    bf = jnp.bfloat16
    g_lo, g_hi = _unpack_halves(g_ref[...])
    k_lo, k_hi = _unpack_halves(k_ref[...])
    emb = jnp.concatenate([g_lo, g_hi, k_lo, k_hi], axis=1)
    h = lax.dot_general(
        emb, w1_ref[...].astype(bf), dnums, preferred_element_type=jnp.float32
    )
    h16 = jnp.maximum(h.astype(bf) + b1_ref[...].astype(bf), jnp.asarray(0, bf))
    o_ref[...] = (
        lax.dot_general(
            h16, w2_ref[...].astype(bf), dnums, preferred_element_type=jnp.float32
        )
        + b2_ref[...]
    )


def _mlp(gbuf, kbuf, w1, w2, b1, b2):
    return pl.pallas_call(
        _mlp_body,
        grid=(N // BLK,),
        in_specs=[
            pl.BlockSpec((BLK, EMBW), lambda i: (i, 0)),
            pl.BlockSpec((BLK, EMBW), lambda i: (i, 0)),
            pl.BlockSpec((H2, 2 * EMB), lambda i: (0, 0)),
            pl.BlockSpec((OUT, H2), lambda i: (0, 0)),
            pl.BlockSpec((1, H2), lambda i: (0, 0)),
            pl.BlockSpec((1, OUT), lambda i: (0, 0)),
        ],
        out_specs=pl.BlockSpec((BLK, OUT), lambda i: (i, 0)),
        out_shape=jax.ShapeDtypeStruct((N, OUT), jnp.float32),
        compiler_params=pltpu.CompilerParams(
            dimension_semantics=("parallel",)
        ),
    )(gbuf, kbuf, w1, w2, b1, b2)


def kernel(input_tensor, genre_table, key_table, W1, b1, W2, b2):
    g_idx = input_tensor[:, 0]
    k_idx = input_tensor[:, 1]
    gw, kw = _pack_tables(genre_table, key_table)
    gbuf, kbuf = _sc_gather(gw, kw, g_idx, k_idx)
    return _mlp(gbuf, kbuf, W1, W2, b1.reshape(1, H2), b2.reshape(1, OUT))
